# baseline (speedup 1.0000x reference)
import functools
import math

import jax
import jax.numpy as jnp
from jax import lax
from jax.experimental import pallas as pl
from jax.experimental.pallas import tpu as pltpu
from jax.experimental.pallas import tpu_sc as plsc

EMB = 64
LANES = 16
SCALE = math.sqrt(EMB)

NC = 2
NS = 16
NW = NC * NS

CHUNK = 128
NBUF = 4


@functools.lru_cache(maxsize=None)
def _build(hist, btiles):
    ngroups = hist * btiles
    gpw = ngroups // NW
    jbits = btiles.bit_length() - 1
    assert (1 << jbits) == btiles
    dtiles = EMB // 8
    mesh = plsc.VectorSubcoreMesh(core_axis_name="c", subcore_axis_name="s")

    @functools.partial(
        pl.kernel,
        mesh=mesh,
        out_type=jax.ShapeDtypeStruct((hist, dtiles, btiles, 8, CHUNK), jnp.float32),
        scratch_types=(
            [pltpu.VMEM((gpw, CHUNK), jnp.int32)]
            + [pltpu.VMEM((CHUNK, EMB), jnp.float32) for _ in range(NBUF)]
            + [pltpu.VMEM((dtiles, 8, CHUNK), jnp.float32) for _ in range(NBUF)]
            + [pltpu.SemaphoreType.DMA for _ in range(2 * NBUF)]
        ),
        compiler_params=pltpu.CompilerParams(
            use_tc_tiling_on_sc=False, needs_layout_passes=False),
    )
    def emb(mask_hbm, table_hbm, out_hbm, idx_v, *rest):
        ins = rest[0:NBUF]
        outs = rest[NBUF:2 * NBUF]
        gsems = rest[2 * NBUF:3 * NBUF]
        osems = rest[3 * NBUF:4 * NBUF]

        cid = lax.axis_index("c")
        sid = lax.axis_index("s")
        wid = sid * NC + cid

        pltpu.sync_copy(mask_hbm.at[wid], idx_v)

        c16 = jax.lax.broadcasted_iota(jnp.int32, (16,), 0)
        rows = [c16 + cb * LANES for cb in range(CHUNK // LANES)]

        def fire_gather(g, b):
            pltpu.async_copy(table_hbm.at[idx_v.at[g]], ins[b], gsems[b])

        def wait_gather(g, b):
            pltpu.make_async_copy(table_hbm.at[idx_v.at[g]], ins[b], gsems[b]).wait()

        def hj(g):
            gid = wid * gpw + g
            return lax.shift_right_logical(gid, jbits), lax.bitwise_and(gid, btiles - 1)

        def fire_out(g, b):
            h, j = hj(g)
            for i in range(dtiles):
                pltpu.async_copy(outs[b].at[i], out_hbm.at[h, i, j], osems[b])

        def wait_out(g, b):
            h, j = hj(g)
            for i in range(dtiles):
                pltpu.make_async_copy(outs[b].at[i], out_hbm.at[h, i, j], osems[b]).wait()

        def transpose_scale(b):
            def dbody(d, _):
                col = jnp.full((16,), d, jnp.int32)
                dhi = lax.shift_right_logical(d, 3)
                dlo = lax.bitwise_and(d, 7)
                for cb in range(CHUNK // LANES):
                    vals = plsc.load_gather(ins[b], [rows[cb], col])
                    outs[b][dhi, dlo, pl.ds(cb * LANES, LANES)] = vals * SCALE
                return 0
            lax.fori_loop(0, EMB, dbody, 0)

        ngrp = gpw // NBUF

        for b in range(NBUF):
            fire_gather(b, b)
        for b in range(NBUF):
            wait_gather(b, b)
            transpose_scale(b)
            fire_out(b, b)
            fire_gather(b + NBUF, b)

        def group(gg, _):
            for b in range(NBUF):
                g = gg * NBUF + b
                wait_gather(g, b)
                wait_out(g - NBUF, b)
                transpose_scale(b)
                fire_out(g, b)
                fire_gather(g + NBUF, b)
            return 0
        lax.fori_loop(1, ngrp - 1, group, 0)

        for b in range(NBUF):
            g = (ngrp - 1) * NBUF + b
            wait_gather(g, b)
            wait_out(g - NBUF, b)
            transpose_scale(b)
            fire_out(g, b)
        for b in range(NBUF):
            g = (ngrp - 1) * NBUF + b
            wait_out(g, b)

    return emb


def kernel(mask, weight):
    bsz, hist = mask.shape
    btiles = bsz // CHUNK
    mask3 = jnp.transpose(mask).astype(jnp.int32).reshape(NW, (hist * btiles) // NW, CHUNK)
    out5 = _build(hist, btiles)(mask3, weight)
    return out5.transpose(2, 4, 0, 1, 3).reshape(bsz, hist, EMB)


# speedup vs baseline: 1.7031x; 1.7031x over previous
import functools
import math

import jax
import jax.numpy as jnp
from jax import lax
from jax.experimental import pallas as pl
from jax.experimental.pallas import tpu as pltpu
from jax.experimental.pallas import tpu_sc as plsc

EMB = 64
LANES = 16
SCALE = math.sqrt(EMB)

NC = 2
NS = 16
NW = NC * NS

CHUNK = 128
NBUF = 4


@functools.lru_cache(maxsize=None)
def _build(hist, btiles):
    ngroups = hist * btiles
    gpw = ngroups // NW
    jbits = btiles.bit_length() - 1
    assert (1 << jbits) == btiles
    dtiles = EMB // 8
    mesh = plsc.VectorSubcoreMesh(core_axis_name="c", subcore_axis_name="s")

    @functools.partial(
        pl.kernel,
        mesh=mesh,
        out_type=jax.ShapeDtypeStruct((hist, dtiles, btiles, 8, CHUNK), jnp.float32),
        scratch_types=(
            [pltpu.VMEM((gpw, CHUNK), jnp.int32)]
            + [pltpu.VMEM((CHUNK, EMB), jnp.float32) for _ in range(NBUF)]
            + [pltpu.VMEM((EMB, CHUNK), jnp.float32) for _ in range(NBUF)]
            + [pltpu.SemaphoreType.DMA for _ in range(2 * NBUF)]
        ),
        compiler_params=pltpu.CompilerParams(
            use_tc_tiling_on_sc=False, needs_layout_passes=False),
    )
    def emb(mask_hbm, table_hbm, out_hbm, idx_v, *rest):
        ins = rest[0:NBUF]
        outs = rest[NBUF:2 * NBUF]
        gsems = rest[2 * NBUF:3 * NBUF]
        osems = rest[3 * NBUF:4 * NBUF]

        cid = lax.axis_index("c")
        sid = lax.axis_index("s")
        wid = sid * NC + cid

        pltpu.sync_copy(mask_hbm.at[wid], idx_v)

        c16 = jax.lax.broadcasted_iota(jnp.int32, (16,), 0)
        diag = [lax.bitwise_and(c16 + d0, 15) for d0 in range(LANES)]

        def fire_gather(g, b):
            pltpu.async_copy(table_hbm.at[idx_v.at[g]], ins[b], gsems[b])

        def wait_gather(g, b):
            pltpu.make_async_copy(table_hbm.at[idx_v.at[g]], ins[b], gsems[b]).wait()

        def hj(g):
            gid = wid * gpw + g
            return lax.shift_right_logical(gid, jbits), lax.bitwise_and(gid, btiles - 1)

        def fire_out(g, b):
            h, j = hj(g)
            for i in range(dtiles):
                pltpu.async_copy(outs[b].at[pl.ds(i * 8, 8)], out_hbm.at[h, i, j],
                                 osems[b])

        def wait_out(g, b):
            h, j = hj(g)
            for i in range(dtiles):
                pltpu.make_async_copy(outs[b].at[pl.ds(i * 8, 8)],
                                      out_hbm.at[h, i, j], osems[b]).wait()

        def transpose_scale(b):
            def blkbody(blk, _):
                c0 = lax.shift_left(lax.shift_right_logical(blk, 2), 4)
                dbase = lax.shift_left(lax.bitwise_and(blk, 3), 4)
                row = c16 + c0
                for d0 in range(LANES):
                    col = diag[d0] + dbase
                    vals = plsc.load_gather(ins[b], [row, col])
                    plsc.store_scatter(outs[b], [col, row], vals * SCALE)
                return 0
            lax.fori_loop(0, (CHUNK // LANES) * (EMB // LANES), blkbody, 0)

        ngrp = gpw // NBUF

        for b in range(NBUF):
            fire_gather(b, b)
        for b in range(NBUF):
            wait_gather(b, b)
            transpose_scale(b)
            fire_out(b, b)
            fire_gather(b + NBUF, b)

        def group(gg, _):
            for b in range(NBUF):
                g = gg * NBUF + b
                wait_gather(g, b)
                wait_out(g - NBUF, b)
                transpose_scale(b)
                fire_out(g, b)
                fire_gather(g + NBUF, b)
            return 0
        lax.fori_loop(1, ngrp - 1, group, 0)

        for b in range(NBUF):
            g = (ngrp - 1) * NBUF + b
            wait_gather(g, b)
            wait_out(g - NBUF, b)
            transpose_scale(b)
            fire_out(g, b)
        for b in range(NBUF):
            g = (ngrp - 1) * NBUF + b
            wait_out(g, b)

    return emb


def kernel(mask, weight):
    bsz, hist = mask.shape
    btiles = bsz // CHUNK
    mask3 = jnp.transpose(mask).astype(jnp.int32).reshape(NW, (hist * btiles) // NW, CHUNK)
    out5 = _build(hist, btiles)(mask3, weight)
    return out5.transpose(2, 4, 0, 1, 3).reshape(bsz, hist, EMB)


# speedup vs baseline: 1.7619x; 1.0345x over previous
import functools
import math

import jax
import jax.numpy as jnp
from jax import lax
from jax.experimental import pallas as pl
from jax.experimental.pallas import tpu as pltpu
from jax.experimental.pallas import tpu_sc as plsc

EMB = 64
LANES = 16
SCALE = math.sqrt(EMB)

NC = 2
NS = 16
NW = NC * NS

CHUNK = 128
NBUF = 5


@functools.lru_cache(maxsize=None)
def _build(hist, btiles):
    ngroups = hist * btiles
    gpw = ngroups // NW
    jbits = btiles.bit_length() - 1
    assert (1 << jbits) == btiles
    dtiles = EMB // 8
    mesh = plsc.VectorSubcoreMesh(core_axis_name="c", subcore_axis_name="s")

    @functools.partial(
        pl.kernel,
        mesh=mesh,
        out_type=jax.ShapeDtypeStruct((hist, dtiles, btiles, 8, CHUNK), jnp.float32),
        scratch_types=(
            [pltpu.VMEM((gpw, CHUNK), jnp.int32)]
            + [pltpu.VMEM((CHUNK, EMB), jnp.float32) for _ in range(NBUF)]
            + [pltpu.VMEM((dtiles, 8, CHUNK), jnp.float32) for _ in range(NBUF)]
            + [pltpu.SemaphoreType.DMA for _ in range(2 * NBUF)]
        ),
        compiler_params=pltpu.CompilerParams(
            use_tc_tiling_on_sc=False, needs_layout_passes=False),
    )
    def emb(mask_hbm, table_hbm, out_hbm, idx_v, *rest):
        ins = rest[0:NBUF]
        outs = rest[NBUF:2 * NBUF]
        gsems = rest[2 * NBUF:3 * NBUF]
        osems = rest[3 * NBUF:4 * NBUF]

        cid = lax.axis_index("c")
        sid = lax.axis_index("s")
        wid = sid * NC + cid

        pltpu.sync_copy(mask_hbm.at[wid], idx_v)

        c16 = jax.lax.broadcasted_iota(jnp.int32, (16,), 0)
        diag = [lax.bitwise_and(c16 + d0, 15) for d0 in range(LANES)]

        def fire_gather(g, b):
            pltpu.async_copy(table_hbm.at[idx_v.at[g]], ins[b], gsems[b])

        def wait_gather(g, b):
            pltpu.make_async_copy(table_hbm.at[idx_v.at[g]], ins[b], gsems[b]).wait()

        def hj(g):
            gid = wid * gpw + g
            return lax.shift_right_logical(gid, jbits), lax.bitwise_and(gid, btiles - 1)

        def fire_out(g, b):
            h, j = hj(g)
            pltpu.async_copy(outs[b], out_hbm.at[h, :, j], osems[b])

        def wait_out(g, b):
            h, j = hj(g)
            pltpu.make_async_copy(outs[b], out_hbm.at[h, :, j], osems[b]).wait()

        def transpose_scale(b):
            def blkbody(blk, _):
                c0 = lax.shift_left(lax.shift_right_logical(blk, 2), 4)
                dbase = lax.shift_left(lax.bitwise_and(blk, 3), 4)
                row = c16 + c0
                for d0 in range(LANES):
                    col = diag[d0] + dbase
                    vals = plsc.load_gather(ins[b], [row, col])
                    plsc.store_scatter(
                        outs[b],
                        [lax.shift_right_logical(col, 3), lax.bitwise_and(col, 7), row],
                        vals * SCALE)
                return 0
            lax.fori_loop(0, (CHUNK // LANES) * (EMB // LANES), blkbody, 0)

        ngrp = gpw // NBUF

        for b in range(NBUF):
            fire_gather(b, b)
        for b in range(NBUF):
            wait_gather(b, b)
            transpose_scale(b)
            fire_out(b, b)
            fire_gather(b + NBUF, b)

        def group(gg, _):
            for b in range(NBUF):
                g = gg * NBUF + b
                wait_gather(g, b)
                wait_out(g - NBUF, b)
                transpose_scale(b)
                fire_out(g, b)
                fire_gather(g + NBUF, b)
            return 0
        lax.fori_loop(1, ngrp - 1, group, 0)

        for b in range(NBUF):
            g = (ngrp - 1) * NBUF + b
            wait_gather(g, b)
            wait_out(g - NBUF, b)
            transpose_scale(b)
            fire_out(g, b)
        for b in range(NBUF):
            g = (ngrp - 1) * NBUF + b
            wait_out(g, b)

    return emb


def kernel(mask, weight):
    bsz, hist = mask.shape
    btiles = bsz // CHUNK
    mask3 = jnp.transpose(mask).astype(jnp.int32).reshape(NW, (hist * btiles) // NW, CHUNK)
    out5 = _build(hist, btiles)(mask3, weight)
    return out5.transpose(2, 4, 0, 1, 3).reshape(bsz, hist, EMB)


# speedup vs baseline: 1.7731x; 1.0064x over previous
import functools
import math

import jax
import jax.numpy as jnp
from jax import lax
from jax.experimental import pallas as pl
from jax.experimental.pallas import tpu as pltpu
from jax.experimental.pallas import tpu_sc as plsc

EMB = 64
LANES = 16
SCALE = math.sqrt(EMB)

NC = 2
NS = 16
NW = NC * NS

CHUNK = 128
NBUF = 5


@functools.lru_cache(maxsize=None)
def _build_prep(vocab):
    vtiles = (vocab + CHUNK - 1) // CHUNK
    jmain = (vtiles // NW) * NW
    jpw = jmain // NW
    nextra = vtiles - jmain
    nb = 4
    assert jpw % nb == 0
    mesh = plsc.VectorSubcoreMesh(core_axis_name="c", subcore_axis_name="s")

    @functools.partial(
        pl.kernel,
        mesh=mesh,
        out_type=jax.ShapeDtypeStruct((vocab // 2, CHUNK), jnp.float32),
        scratch_types=(
            [pltpu.VMEM((EMB, CHUNK), jnp.float32) for _ in range(2 * nb)]
            + [pltpu.SemaphoreType.DMA for _ in range(2 * nb)]
        ),
        compiler_params=pltpu.CompilerParams(
            use_tc_tiling_on_sc=True, needs_layout_passes=False),
    )
    def prep(wt_hbm, out_hbm, *rest):
        ins = rest[0:nb]
        outs = rest[nb:2 * nb]
        gsems = rest[2 * nb:3 * nb]
        osems = rest[3 * nb:4 * nb]

        cid = lax.axis_index("c")
        sid = lax.axis_index("s")
        wid = sid * NC + cid

        c16 = jax.lax.broadcasted_iota(jnp.int32, (16,), 0)
        diag = [lax.bitwise_and(c16 + t, 15) for t in range(LANES)]
        c16h = lax.shift_right_logical(c16, 1)
        pars = lax.shift_left(lax.bitwise_and(c16, 1), 6)

        def fire_in(jj, b):
            v0 = jj * CHUNK
            for i in range(EMB // 8):
                pltpu.async_copy(
                    wt_hbm.at[pl.ds(i * 8, 8), pl.ds(v0, CHUNK)],
                    ins[b].at[pl.ds(i * 8, 8)], gsems[b])

        def wait_in(jj, b):
            v0 = jj * CHUNK
            for i in range(EMB // 8):
                pltpu.make_async_copy(
                    wt_hbm.at[pl.ds(i * 8, 8), pl.ds(v0, CHUNK)],
                    ins[b].at[pl.ds(i * 8, 8)], gsems[b]).wait()

        def fire_out(jj, b):
            pltpu.async_copy(outs[b], out_hbm.at[pl.ds(jj * EMB, EMB)], osems[b])

        def wait_out(jj, b):
            pltpu.make_async_copy(
                outs[b], out_hbm.at[pl.ds(jj * EMB, EMB)], osems[b]).wait()

        def transpose_block(b):
            def blkbody(blk, _):
                vv0 = lax.shift_left(lax.shift_right_logical(blk, 2), 4)
                dbase = lax.shift_left(lax.bitwise_and(blk, 3), 4)
                vv = c16 + vv0
                r = c16h + lax.shift_right_logical(vv0, 1)
                for t in range(LANES):
                    d = diag[t] + dbase
                    vals = plsc.load_gather(ins[b], [d, vv])
                    plsc.store_scatter(outs[b], [r, pars + d], vals * SCALE)
                return 0
            lax.fori_loop(0, (CHUNK // LANES) * (EMB // LANES), blkbody, 0)

        def jglob(j):
            return wid * jpw + j

        for b in range(nb):
            fire_in(jglob(b), b)
        for b in range(nb):
            wait_in(jglob(b), b)
            transpose_block(b)
            fire_out(jglob(b), b)
            fire_in(jglob(b + nb), b)

        def group(gg, _):
            for b in range(nb):
                j = gg * nb + b
                wait_in(jglob(j), b)
                wait_out(jglob(j - nb), b)
                transpose_block(b)
                fire_out(jglob(j), b)
                fire_in(jglob(j + nb), b)
            return 0
        lax.fori_loop(1, jpw // nb - 1, group, 0)

        for b in range(nb):
            j = (jpw // nb - 1) * nb + b
            wait_in(jglob(j), b)
            wait_out(jglob(j - nb), b)
            transpose_block(b)
            fire_out(jglob(j), b)
        for b in range(nb):
            j = (jpw // nb - 1) * nb + b
            wait_out(jglob(j), b)

        half_last = (vocab % CHUNK) != 0
        if nextra:
            nfull = nextra - 1 if half_last else nextra

            @pl.when(wid < nfull)
            def _tail_full():
                j = jmain + wid
                fire_in(j, 0)
                wait_in(j, 0)
                transpose_block(0)
                pltpu.sync_copy(outs[0], out_hbm.at[pl.ds(j * EMB, EMB)])


    return prep


@functools.lru_cache(maxsize=None)
def _build(hist, btiles):
    ngroups = hist * btiles
    gpw = ngroups // NW
    jbits = btiles.bit_length() - 1
    assert (1 << jbits) == btiles
    dtiles = EMB // 8
    mesh = plsc.VectorSubcoreMesh(core_axis_name="c", subcore_axis_name="s")

    @functools.partial(
        pl.kernel,
        mesh=mesh,
        out_type=jax.ShapeDtypeStruct((hist, dtiles, btiles, 8, CHUNK), jnp.float32),
        scratch_types=(
            [pltpu.VMEM((gpw, CHUNK), jnp.int32)]
            + [pltpu.VMEM((CHUNK, EMB), jnp.float32) for _ in range(NBUF)]
            + [pltpu.VMEM((dtiles, 8, CHUNK), jnp.float32) for _ in range(NBUF)]
            + [pltpu.SemaphoreType.DMA for _ in range(2 * NBUF)]
        ),
        compiler_params=pltpu.CompilerParams(
            use_tc_tiling_on_sc=False, needs_layout_passes=False),
    )
    def emb(mask_hbm, table_hbm, out_hbm, idx_v, *rest):
        ins = rest[0:NBUF]
        outs = rest[NBUF:2 * NBUF]
        gsems = rest[2 * NBUF:3 * NBUF]
        osems = rest[3 * NBUF:4 * NBUF]

        cid = lax.axis_index("c")
        sid = lax.axis_index("s")
        wid = sid * NC + cid

        pltpu.sync_copy(mask_hbm.at[wid], idx_v)

        c16 = jax.lax.broadcasted_iota(jnp.int32, (16,), 0)
        diag = [lax.bitwise_and(c16 + d0, 15) for d0 in range(LANES)]

        def fire_gather(g, b):
            pltpu.async_copy(table_hbm.at[idx_v.at[g]], ins[b], gsems[b])

        def wait_gather(g, b):
            pltpu.make_async_copy(table_hbm.at[idx_v.at[g]], ins[b], gsems[b]).wait()

        def hj(g):
            gid = wid * gpw + g
            return lax.shift_right_logical(gid, jbits), lax.bitwise_and(gid, btiles - 1)

        def fire_out(g, b):
            h, j = hj(g)
            pltpu.async_copy(outs[b], out_hbm.at[h, :, j], osems[b])

        def wait_out(g, b):
            h, j = hj(g)
            pltpu.make_async_copy(outs[b], out_hbm.at[h, :, j], osems[b]).wait()

        def transpose_scale(b):
            def blkbody(blk, _):
                c0 = lax.shift_left(lax.shift_right_logical(blk, 2), 4)
                dbase = lax.shift_left(lax.bitwise_and(blk, 3), 4)
                row = c16 + c0
                for d0 in range(LANES):
                    col = diag[d0] + dbase
                    vals = plsc.load_gather(ins[b], [row, col])
                    plsc.store_scatter(
                        outs[b],
                        [lax.shift_right_logical(col, 3), lax.bitwise_and(col, 7), row],
                        vals)
                return 0
            lax.fori_loop(0, (CHUNK // LANES) * (EMB // LANES), blkbody, 0)

        ngrp = gpw // NBUF

        for b in range(NBUF):
            fire_gather(b, b)
        for b in range(NBUF):
            wait_gather(b, b)
            transpose_scale(b)
            fire_out(b, b)
            fire_gather(b + NBUF, b)

        def group(gg, _):
            for b in range(NBUF):
                g = gg * NBUF + b
                wait_gather(g, b)
                wait_out(g - NBUF, b)
                transpose_scale(b)
                fire_out(g, b)
                fire_gather(g + NBUF, b)
            return 0
        lax.fori_loop(1, ngrp - 1, group, 0)

        for b in range(NBUF):
            g = (ngrp - 1) * NBUF + b
            wait_gather(g, b)
            wait_out(g - NBUF, b)
            transpose_scale(b)
            fire_out(g, b)
        for b in range(NBUF):
            g = (ngrp - 1) * NBUF + b
            wait_out(g, b)

    return emb


def kernel(mask, weight):
    bsz, hist = mask.shape
    btiles = bsz // CHUNK
    mask3 = jnp.transpose(mask).astype(jnp.int32).reshape(NW, (hist * btiles) // NW, CHUNK)
    vocab = weight.shape[0]
    wt2 = _build_prep(vocab)(jnp.transpose(weight))
    if vocab % CHUNK:
        vtail = (vocab // CHUNK) * CHUNK
        tail = (weight[vtail:] * jnp.float32(SCALE)).reshape(-1, CHUNK)
        wt2 = jax.lax.dynamic_update_slice(wt2, tail, (vtail * EMB // CHUNK, 0))
    out5 = _build(hist, btiles)(mask3, wt2.reshape(vocab, EMB))
    return out5.transpose(2, 4, 0, 1, 3).reshape(bsz, hist, EMB)


# speedup vs baseline: 3.8042x; 2.1454x over previous
import functools
import math

import jax
import jax.numpy as jnp
from jax import lax
from jax.experimental import pallas as pl
from jax.experimental.pallas import tpu as pltpu
from jax.experimental.pallas import tpu_sc as plsc

EMB = 64
LANES = 16
SCALE = math.sqrt(EMB)

NC = 2
NS = 16
NW = NC * NS

CHUNK = 128
NBUF = 5


@functools.lru_cache(maxsize=None)
def _build_prep(vocab):
    vtiles = (vocab + CHUNK - 1) // CHUNK
    jmain = (vtiles // NW) * NW
    jpw = jmain // NW
    nextra = vtiles - jmain
    nb = 4
    assert jpw % nb == 0
    mesh = plsc.VectorSubcoreMesh(core_axis_name="c", subcore_axis_name="s")

    @functools.partial(
        pl.kernel,
        mesh=mesh,
        out_type=jax.ShapeDtypeStruct((vocab // 2, CHUNK), jnp.float32),
        scratch_types=(
            [pltpu.VMEM((EMB, CHUNK), jnp.float32) for _ in range(2 * nb)]
            + [pltpu.SemaphoreType.DMA for _ in range(2 * nb)]
        ),
        compiler_params=pltpu.CompilerParams(
            use_tc_tiling_on_sc=True, needs_layout_passes=False),
    )
    def prep(wt_hbm, out_hbm, *rest):
        ins = rest[0:nb]
        outs = rest[nb:2 * nb]
        gsems = rest[2 * nb:3 * nb]
        osems = rest[3 * nb:4 * nb]

        cid = lax.axis_index("c")
        sid = lax.axis_index("s")
        wid = sid * NC + cid

        c16 = jax.lax.broadcasted_iota(jnp.int32, (16,), 0)
        diag = [lax.bitwise_and(c16 + t, 15) for t in range(LANES)]
        c16h = lax.shift_right_logical(c16, 1)
        pars = lax.shift_left(lax.bitwise_and(c16, 1), 6)

        def fire_in(jj, b):
            v0 = jj * CHUNK
            for i in range(EMB // 8):
                pltpu.async_copy(
                    wt_hbm.at[pl.ds(i * 8, 8), pl.ds(v0, CHUNK)],
                    ins[b].at[pl.ds(i * 8, 8)], gsems[b])

        def wait_in(jj, b):
            v0 = jj * CHUNK
            for i in range(EMB // 8):
                pltpu.make_async_copy(
                    wt_hbm.at[pl.ds(i * 8, 8), pl.ds(v0, CHUNK)],
                    ins[b].at[pl.ds(i * 8, 8)], gsems[b]).wait()

        def fire_out(jj, b):
            pltpu.async_copy(outs[b], out_hbm.at[pl.ds(jj * EMB, EMB)], osems[b])

        def wait_out(jj, b):
            pltpu.make_async_copy(
                outs[b], out_hbm.at[pl.ds(jj * EMB, EMB)], osems[b]).wait()

        def transpose_block(b):
            def blkbody(blk, _):
                vv0 = lax.shift_left(lax.shift_right_logical(blk, 2), 4)
                dbase = lax.shift_left(lax.bitwise_and(blk, 3), 4)
                vv = c16 + vv0
                r = c16h + lax.shift_right_logical(vv0, 1)
                ds_ = [diag[t] + dbase for t in range(LANES)]
                vals = [plsc.load_gather(ins[b], [ds_[t], vv]) for t in range(LANES)]
                for t in range(LANES):
                    plsc.store_scatter(outs[b], [r, pars + ds_[t]], vals[t] * SCALE)
                return 0
            lax.fori_loop(0, (CHUNK // LANES) * (EMB // LANES), blkbody, 0)

        def jglob(j):
            return wid * jpw + j

        for b in range(nb):
            fire_in(jglob(b), b)
        for b in range(nb):
            wait_in(jglob(b), b)
            transpose_block(b)
            fire_out(jglob(b), b)
            fire_in(jglob(b + nb), b)

        def group(gg, _):
            for b in range(nb):
                j = gg * nb + b
                wait_in(jglob(j), b)
                wait_out(jglob(j - nb), b)
                transpose_block(b)
                fire_out(jglob(j), b)
                fire_in(jglob(j + nb), b)
            return 0
        lax.fori_loop(1, jpw // nb - 1, group, 0)

        for b in range(nb):
            j = (jpw // nb - 1) * nb + b
            wait_in(jglob(j), b)
            wait_out(jglob(j - nb), b)
            transpose_block(b)
            fire_out(jglob(j), b)
        for b in range(nb):
            j = (jpw // nb - 1) * nb + b
            wait_out(jglob(j), b)

        half_last = (vocab % CHUNK) != 0
        if nextra:
            nfull = nextra - 1 if half_last else nextra

            @pl.when(wid < nfull)
            def _tail_full():
                j = jmain + wid
                fire_in(j, 0)
                wait_in(j, 0)
                transpose_block(0)
                pltpu.sync_copy(outs[0], out_hbm.at[pl.ds(j * EMB, EMB)])


    return prep


@functools.lru_cache(maxsize=None)
def _build(hist, btiles):
    ngroups = hist * btiles
    gpw = ngroups // NW
    jbits = btiles.bit_length() - 1
    assert (1 << jbits) == btiles
    dtiles = EMB // 8
    mesh = plsc.VectorSubcoreMesh(core_axis_name="c", subcore_axis_name="s")

    @functools.partial(
        pl.kernel,
        mesh=mesh,
        out_type=jax.ShapeDtypeStruct((hist, dtiles, btiles, 8, CHUNK), jnp.float32),
        scratch_types=(
            [pltpu.VMEM((gpw, CHUNK), jnp.int32)]
            + [pltpu.VMEM((CHUNK, EMB), jnp.float32) for _ in range(NBUF)]
            + [pltpu.VMEM((dtiles, 8, CHUNK), jnp.float32) for _ in range(NBUF)]
            + [pltpu.SemaphoreType.DMA for _ in range(2 * NBUF)]
        ),
        compiler_params=pltpu.CompilerParams(
            use_tc_tiling_on_sc=False, needs_layout_passes=False),
    )
    def emb(mask_hbm, table_hbm, out_hbm, idx_v, *rest):
        ins = rest[0:NBUF]
        outs = rest[NBUF:2 * NBUF]
        gsems = rest[2 * NBUF:3 * NBUF]
        osems = rest[3 * NBUF:4 * NBUF]

        cid = lax.axis_index("c")
        sid = lax.axis_index("s")
        wid = sid * NC + cid

        pltpu.sync_copy(mask_hbm.at[wid], idx_v)

        c16 = jax.lax.broadcasted_iota(jnp.int32, (16,), 0)
        diag = [lax.bitwise_and(c16 + d0, 15) for d0 in range(LANES)]

        def fire_gather(g, b):
            pltpu.async_copy(table_hbm.at[idx_v.at[g]], ins[b], gsems[b])

        def wait_gather(g, b):
            pltpu.make_async_copy(table_hbm.at[idx_v.at[g]], ins[b], gsems[b]).wait()

        def hj(g):
            gid = wid * gpw + g
            return lax.shift_right_logical(gid, jbits), lax.bitwise_and(gid, btiles - 1)

        def fire_out(g, b):
            h, j = hj(g)
            pltpu.async_copy(outs[b], out_hbm.at[h, :, j], osems[b])

        def wait_out(g, b):
            h, j = hj(g)
            pltpu.make_async_copy(outs[b], out_hbm.at[h, :, j], osems[b]).wait()

        def transpose_scale(b):
            def blkbody(blk, _):
                c0 = lax.shift_left(lax.shift_right_logical(blk, 2), 4)
                dbase = lax.shift_left(lax.bitwise_and(blk, 3), 4)
                row = c16 + c0
                cols = [diag[d0] + dbase for d0 in range(LANES)]
                vals = [plsc.load_gather(ins[b], [row, cols[d0]])
                        for d0 in range(LANES)]
                for d0 in range(LANES):
                    col = cols[d0]
                    plsc.store_scatter(
                        outs[b],
                        [lax.shift_right_logical(col, 3), lax.bitwise_and(col, 7), row],
                        vals[d0])
                return 0
            lax.fori_loop(0, (CHUNK // LANES) * (EMB // LANES), blkbody, 0)

        ngrp = gpw // NBUF

        for b in range(NBUF):
            fire_gather(b, b)
        for b in range(NBUF):
            wait_gather(b, b)
            transpose_scale(b)
            fire_out(b, b)
            fire_gather(b + NBUF, b)

        def group(gg, _):
            for b in range(NBUF):
                g = gg * NBUF + b
                wait_gather(g, b)
                wait_out(g - NBUF, b)
                transpose_scale(b)
                fire_out(g, b)
                fire_gather(g + NBUF, b)
            return 0
        lax.fori_loop(1, ngrp - 1, group, 0)

        for b in range(NBUF):
            g = (ngrp - 1) * NBUF + b
            wait_gather(g, b)
            wait_out(g - NBUF, b)
            transpose_scale(b)
            fire_out(g, b)
        for b in range(NBUF):
            g = (ngrp - 1) * NBUF + b
            wait_out(g, b)

    return emb


def kernel(mask, weight):
    bsz, hist = mask.shape
    btiles = bsz // CHUNK
    mask3 = jnp.transpose(mask).astype(jnp.int32).reshape(NW, (hist * btiles) // NW, CHUNK)
    vocab = weight.shape[0]
    wt2 = _build_prep(vocab)(jnp.transpose(weight))
    if vocab % CHUNK:
        vtail = (vocab // CHUNK) * CHUNK
        tail = (weight[vtail:] * jnp.float32(SCALE)).reshape(-1, CHUNK)
        wt2 = jax.lax.dynamic_update_slice(wt2, tail, (vtail * EMB // CHUNK, 0))
    out5 = _build(hist, btiles)(mask3, wt2.reshape(vocab, EMB))
    return out5.transpose(2, 4, 0, 1, 3).reshape(bsz, hist, EMB)


# speedup vs baseline: 5.1441x; 1.3522x over previous
import functools
import math

import jax
import jax.numpy as jnp
from jax import lax
from jax.experimental import pallas as pl
from jax.experimental.pallas import tpu as pltpu
from jax.experimental.pallas import tpu_sc as plsc

EMB = 64
LANES = 16
SCALE = math.sqrt(EMB)

NC = 2
NS = 16
NW = NC * NS

CHUNK = 128
NBUF = 5


@functools.lru_cache(maxsize=None)
def _build_prep(vocab):
    vtiles = (vocab + CHUNK - 1) // CHUNK
    jmain = (vtiles // NW) * NW
    jpw = jmain // NW
    nextra = vtiles - jmain
    nb = 4
    assert jpw % nb == 0
    mesh = plsc.VectorSubcoreMesh(core_axis_name="c", subcore_axis_name="s")

    @functools.partial(
        pl.kernel,
        mesh=mesh,
        out_type=jax.ShapeDtypeStruct((vocab // 2 * CHUNK,), jnp.float32),
        scratch_types=(
            [pltpu.VMEM((EMB, CHUNK), jnp.float32) for _ in range(nb)]
            + [pltpu.VMEM((EMB * CHUNK,), jnp.float32) for _ in range(nb)]
            + [pltpu.SemaphoreType.DMA for _ in range(2 * nb)]
        ),
        compiler_params=pltpu.CompilerParams(
            use_tc_tiling_on_sc=True, needs_layout_passes=False),
    )
    def prep(wt_hbm, out_hbm, *rest):
        ins = rest[0:nb]
        outs = rest[nb:2 * nb]
        gsems = rest[2 * nb:3 * nb]
        osems = rest[3 * nb:4 * nb]

        cid = lax.axis_index("c")
        sid = lax.axis_index("s")
        wid = sid * NC + cid

        c16 = jax.lax.broadcasted_iota(jnp.int32, (16,), 0)
        diag = [lax.bitwise_and(c16 + t, 15) for t in range(LANES)]
        c16h = lax.shift_right_logical(c16, 1)
        pars = lax.shift_left(lax.bitwise_and(c16, 1), 6)
        sdiag = [lax.shift_left(c16h, 7) + pars + diag[t] for t in range(LANES)]

        def fire_in(jj, b):
            v0 = jj * CHUNK
            for i in range(EMB // 8):
                pltpu.async_copy(
                    wt_hbm.at[pl.ds(i * 8, 8), pl.ds(v0, CHUNK)],
                    ins[b].at[pl.ds(i * 8, 8)], gsems[b])

        def wait_in(jj, b):
            v0 = jj * CHUNK
            for i in range(EMB // 8):
                pltpu.make_async_copy(
                    wt_hbm.at[pl.ds(i * 8, 8), pl.ds(v0, CHUNK)],
                    ins[b].at[pl.ds(i * 8, 8)], gsems[b]).wait()

        def fire_out(jj, b):
            pltpu.async_copy(outs[b], out_hbm.at[pl.ds(jj * EMB * CHUNK, EMB * CHUNK)],
                             osems[b])

        def wait_out(jj, b):
            pltpu.make_async_copy(
                outs[b], out_hbm.at[pl.ds(jj * EMB * CHUNK, EMB * CHUNK)],
                osems[b]).wait()

        def transpose_block(b):
            def blkbody(blk, _):
                vv0 = lax.shift_left(lax.shift_right_logical(blk, 2), 4)
                dbase = lax.shift_left(lax.bitwise_and(blk, 3), 4)
                vv = c16 + vv0
                obase = lax.shift_left(vv0, 6) + dbase
                ds_ = [diag[t] + dbase for t in range(LANES)]
                vals = [plsc.load_gather(ins[b], [ds_[t], vv]) for t in range(LANES)]
                for t in range(LANES):
                    plsc.store_scatter(outs[b], [sdiag[t] + obase], vals[t] * SCALE)
                return 0
            lax.fori_loop(0, (CHUNK // LANES) * (EMB // LANES), blkbody, 0)

        def jglob(j):
            return wid * jpw + j

        for b in range(nb):
            fire_in(jglob(b), b)
        for b in range(nb):
            wait_in(jglob(b), b)
            transpose_block(b)
            fire_out(jglob(b), b)
            fire_in(jglob(b + nb), b)

        def group(gg, _):
            for b in range(nb):
                j = gg * nb + b
                wait_in(jglob(j), b)
                wait_out(jglob(j - nb), b)
                transpose_block(b)
                fire_out(jglob(j), b)
                fire_in(jglob(j + nb), b)
            return 0
        lax.fori_loop(1, jpw // nb - 1, group, 0)

        for b in range(nb):
            j = (jpw // nb - 1) * nb + b
            wait_in(jglob(j), b)
            wait_out(jglob(j - nb), b)
            transpose_block(b)
            fire_out(jglob(j), b)
        for b in range(nb):
            j = (jpw // nb - 1) * nb + b
            wait_out(jglob(j), b)

        half_last = (vocab % CHUNK) != 0
        if nextra:
            nfull = nextra - 1 if half_last else nextra

            @pl.when(wid < nfull)
            def _tail_full():
                j = jmain + wid
                fire_in(j, 0)
                wait_in(j, 0)
                transpose_block(0)
                pltpu.sync_copy(outs[0],
                                out_hbm.at[pl.ds(j * EMB * CHUNK, EMB * CHUNK)])


    return prep


@functools.lru_cache(maxsize=None)
def _build(hist, btiles):
    ngroups = hist * btiles
    gpw = ngroups // NW
    jbits = btiles.bit_length() - 1
    assert (1 << jbits) == btiles
    dtiles = EMB // 8
    mesh = plsc.VectorSubcoreMesh(core_axis_name="c", subcore_axis_name="s")

    @functools.partial(
        pl.kernel,
        mesh=mesh,
        out_type=jax.ShapeDtypeStruct((hist, dtiles, btiles, 8, CHUNK), jnp.float32),
        scratch_types=(
            [pltpu.VMEM((gpw, CHUNK), jnp.int32)]
            + [pltpu.VMEM((CHUNK, EMB), jnp.float32) for _ in range(NBUF)]
            + [pltpu.VMEM((dtiles, 8, CHUNK), jnp.float32) for _ in range(NBUF)]
            + [pltpu.SemaphoreType.DMA for _ in range(2 * NBUF)]
        ),
        compiler_params=pltpu.CompilerParams(
            use_tc_tiling_on_sc=False, needs_layout_passes=False),
    )
    def emb(mask_hbm, table_hbm, out_hbm, idx_v, *rest):
        ins = rest[0:NBUF]
        outs = rest[NBUF:2 * NBUF]
        gsems = rest[2 * NBUF:3 * NBUF]
        osems = rest[3 * NBUF:4 * NBUF]

        cid = lax.axis_index("c")
        sid = lax.axis_index("s")
        wid = sid * NC + cid

        pltpu.sync_copy(mask_hbm.at[wid], idx_v)

        c16 = jax.lax.broadcasted_iota(jnp.int32, (16,), 0)
        diag = [lax.bitwise_and(c16 + d0, 15) for d0 in range(LANES)]

        def fire_gather(g, b):
            pltpu.async_copy(table_hbm.at[idx_v.at[g]], ins[b], gsems[b])

        def wait_gather(g, b):
            pltpu.make_async_copy(table_hbm.at[idx_v.at[g]], ins[b], gsems[b]).wait()

        def hj(g):
            gid = wid * gpw + g
            return lax.shift_right_logical(gid, jbits), lax.bitwise_and(gid, btiles - 1)

        def fire_out(g, b):
            h, j = hj(g)
            pltpu.async_copy(outs[b], out_hbm.at[h, :, j], osems[b])

        def wait_out(g, b):
            h, j = hj(g)
            pltpu.make_async_copy(outs[b], out_hbm.at[h, :, j], osems[b]).wait()

        def transpose_scale(b):
            def blkbody(blk, _):
                c0 = lax.shift_left(lax.shift_right_logical(blk, 2), 4)
                dbase = lax.shift_left(lax.bitwise_and(blk, 3), 4)
                row = c16 + c0
                cols = [diag[d0] + dbase for d0 in range(LANES)]
                vals = [plsc.load_gather(ins[b], [row, cols[d0]])
                        for d0 in range(LANES)]
                for d0 in range(LANES):
                    col = cols[d0]
                    plsc.store_scatter(
                        outs[b],
                        [lax.shift_right_logical(col, 3), lax.bitwise_and(col, 7), row],
                        vals[d0])
                return 0
            lax.fori_loop(0, (CHUNK // LANES) * (EMB // LANES), blkbody, 0)

        ngrp = gpw // NBUF

        for b in range(NBUF):
            fire_gather(b, b)
        for b in range(NBUF):
            wait_gather(b, b)
            transpose_scale(b)
            fire_out(b, b)
            fire_gather(b + NBUF, b)

        def group(gg, _):
            for b in range(NBUF):
                g = gg * NBUF + b
                wait_gather(g, b)
                wait_out(g - NBUF, b)
                transpose_scale(b)
                fire_out(g, b)
                fire_gather(g + NBUF, b)
            return 0
        lax.fori_loop(1, ngrp - 1, group, 0)

        for b in range(NBUF):
            g = (ngrp - 1) * NBUF + b
            wait_gather(g, b)
            wait_out(g - NBUF, b)
            transpose_scale(b)
            fire_out(g, b)
        for b in range(NBUF):
            g = (ngrp - 1) * NBUF + b
            wait_out(g, b)

    return emb


def kernel(mask, weight):
    bsz, hist = mask.shape
    btiles = bsz // CHUNK
    mask3 = jnp.transpose(mask).astype(jnp.int32).reshape(NW, (hist * btiles) // NW, CHUNK)
    vocab = weight.shape[0]
    wt2 = _build_prep(vocab)(jnp.transpose(weight))
    if vocab % CHUNK:
        vtail = (vocab // CHUNK) * CHUNK
        tail = (weight[vtail:] * jnp.float32(SCALE)).reshape(-1)
        wt2 = jax.lax.dynamic_update_slice(wt2, tail, (vtail * EMB,))
    out5 = _build(hist, btiles)(mask3, wt2.reshape(vocab, EMB))
    return out5.transpose(2, 4, 0, 1, 3).reshape(bsz, hist, EMB)


# speedup vs baseline: 5.1484x; 1.0008x over previous
import functools
import math

import jax
import jax.numpy as jnp
from jax import lax
from jax.experimental import pallas as pl
from jax.experimental.pallas import tpu as pltpu
from jax.experimental.pallas import tpu_sc as plsc

EMB = 64
LANES = 16
SCALE = math.sqrt(EMB)

NC = 2
NS = 16
NW = NC * NS

CHUNK = 128
NBUF = 5


@functools.lru_cache(maxsize=None)
def _build_prep(vocab):
    vtiles = (vocab + CHUNK - 1) // CHUNK
    jmain = (vtiles // NW) * NW
    jpw = jmain // NW
    nextra = vtiles - jmain
    nb = 4
    assert jpw % nb == 0
    mesh = plsc.VectorSubcoreMesh(core_axis_name="c", subcore_axis_name="s")

    @functools.partial(
        pl.kernel,
        mesh=mesh,
        out_type=jax.ShapeDtypeStruct((vocab // 2 * CHUNK,), jnp.float32),
        scratch_types=(
            [pltpu.VMEM((EMB, CHUNK), jnp.float32) for _ in range(nb)]
            + [pltpu.VMEM((EMB * CHUNK,), jnp.float32) for _ in range(nb)]
            + [pltpu.SemaphoreType.DMA for _ in range(2 * nb)]
        ),
        compiler_params=pltpu.CompilerParams(
            use_tc_tiling_on_sc=True, needs_layout_passes=False),
    )
    def prep(wt_hbm, out_hbm, *rest):
        ins = rest[0:nb]
        outs = rest[nb:2 * nb]
        gsems = rest[2 * nb:3 * nb]
        osems = rest[3 * nb:4 * nb]

        cid = lax.axis_index("c")
        sid = lax.axis_index("s")
        wid = sid * NC + cid

        c16 = jax.lax.broadcasted_iota(jnp.int32, (16,), 0)
        diag = [lax.bitwise_and(c16 + t, 15) for t in range(LANES)]
        c16h = lax.shift_right_logical(c16, 1)
        pars = lax.shift_left(lax.bitwise_and(c16, 1), 6)
        sdiag = [lax.shift_left(c16h, 7) + pars + diag[t] for t in range(LANES)]

        def fire_in(jj, b):
            pltpu.async_copy(wt_hbm.at[:, pl.ds(jj * CHUNK, CHUNK)], ins[b], gsems[b])

        def wait_in(jj, b):
            pltpu.make_async_copy(
                wt_hbm.at[:, pl.ds(jj * CHUNK, CHUNK)], ins[b], gsems[b]).wait()

        def fire_out(jj, b):
            pltpu.async_copy(outs[b], out_hbm.at[pl.ds(jj * EMB * CHUNK, EMB * CHUNK)],
                             osems[b])

        def wait_out(jj, b):
            pltpu.make_async_copy(
                outs[b], out_hbm.at[pl.ds(jj * EMB * CHUNK, EMB * CHUNK)],
                osems[b]).wait()

        def transpose_block(b):
            def blkbody(blk, _):
                vv0 = lax.shift_left(lax.shift_right_logical(blk, 2), 4)
                dbase = lax.shift_left(lax.bitwise_and(blk, 3), 4)
                vv = c16 + vv0
                obase = lax.shift_left(vv0, 6) + dbase
                ds_ = [diag[t] + dbase for t in range(LANES)]
                vals = [plsc.load_gather(ins[b], [ds_[t], vv]) for t in range(LANES)]
                for t in range(LANES):
                    plsc.store_scatter(outs[b], [sdiag[t] + obase], vals[t] * SCALE)
                return 0
            lax.fori_loop(0, (CHUNK // LANES) * (EMB // LANES), blkbody, 0)

        def jglob(j):
            return wid * jpw + j

        for b in range(nb):
            fire_in(jglob(b), b)
        for b in range(nb):
            wait_in(jglob(b), b)
            transpose_block(b)
            fire_out(jglob(b), b)
            fire_in(jglob(b + nb), b)

        def group(gg, _):
            for b in range(nb):
                j = gg * nb + b
                wait_in(jglob(j), b)
                wait_out(jglob(j - nb), b)
                transpose_block(b)
                fire_out(jglob(j), b)
                fire_in(jglob(j + nb), b)
            return 0
        lax.fori_loop(1, jpw // nb - 1, group, 0)

        for b in range(nb):
            j = (jpw // nb - 1) * nb + b
            wait_in(jglob(j), b)
            wait_out(jglob(j - nb), b)
            transpose_block(b)
            fire_out(jglob(j), b)
        for b in range(nb):
            j = (jpw // nb - 1) * nb + b
            wait_out(jglob(j), b)

        half_last = (vocab % CHUNK) != 0
        if nextra:
            nfull = nextra - 1 if half_last else nextra

            @pl.when(wid < nfull)
            def _tail_full():
                j = jmain + wid
                fire_in(j, 0)
                wait_in(j, 0)
                transpose_block(0)
                pltpu.sync_copy(outs[0],
                                out_hbm.at[pl.ds(j * EMB * CHUNK, EMB * CHUNK)])


    return prep


@functools.lru_cache(maxsize=None)
def _build(hist, btiles):
    ngroups = hist * btiles
    gpw = ngroups // NW
    jbits = btiles.bit_length() - 1
    assert (1 << jbits) == btiles
    dtiles = EMB // 8
    mesh = plsc.VectorSubcoreMesh(core_axis_name="c", subcore_axis_name="s")

    @functools.partial(
        pl.kernel,
        mesh=mesh,
        out_type=jax.ShapeDtypeStruct((hist, dtiles, btiles, 8, CHUNK), jnp.float32),
        scratch_types=(
            [pltpu.VMEM((gpw, CHUNK), jnp.int32)]
            + [pltpu.VMEM((CHUNK, EMB), jnp.float32) for _ in range(NBUF)]
            + [pltpu.VMEM((dtiles, 8, CHUNK), jnp.float32) for _ in range(NBUF)]
            + [pltpu.SemaphoreType.DMA for _ in range(2 * NBUF)]
        ),
        compiler_params=pltpu.CompilerParams(
            use_tc_tiling_on_sc=False, needs_layout_passes=False),
    )
    def emb(mask_hbm, table_hbm, out_hbm, idx_v, *rest):
        ins = rest[0:NBUF]
        outs = rest[NBUF:2 * NBUF]
        gsems = rest[2 * NBUF:3 * NBUF]
        osems = rest[3 * NBUF:4 * NBUF]

        cid = lax.axis_index("c")
        sid = lax.axis_index("s")
        wid = sid * NC + cid

        pltpu.sync_copy(mask_hbm.at[wid], idx_v)

        c16 = jax.lax.broadcasted_iota(jnp.int32, (16,), 0)
        diag = [lax.bitwise_and(c16 + d0, 15) for d0 in range(LANES)]

        def fire_gather(g, b):
            pltpu.async_copy(table_hbm.at[idx_v.at[g]], ins[b], gsems[b])

        def wait_gather(g, b):
            pltpu.make_async_copy(table_hbm.at[idx_v.at[g]], ins[b], gsems[b]).wait()

        def hj(g):
            gid = wid * gpw + g
            return lax.shift_right_logical(gid, jbits), lax.bitwise_and(gid, btiles - 1)

        def fire_out(g, b):
            h, j = hj(g)
            pltpu.async_copy(outs[b], out_hbm.at[h, :, j], osems[b])

        def wait_out(g, b):
            h, j = hj(g)
            pltpu.make_async_copy(outs[b], out_hbm.at[h, :, j], osems[b]).wait()

        def transpose_scale(b):
            def blkbody(blk, _):
                c0 = lax.shift_left(lax.shift_right_logical(blk, 2), 4)
                dbase = lax.shift_left(lax.bitwise_and(blk, 3), 4)
                row = c16 + c0
                cols = [diag[d0] + dbase for d0 in range(LANES)]
                vals = [plsc.load_gather(ins[b], [row, cols[d0]])
                        for d0 in range(LANES)]
                for d0 in range(LANES):
                    col = cols[d0]
                    plsc.store_scatter(
                        outs[b],
                        [lax.shift_right_logical(col, 3), lax.bitwise_and(col, 7), row],
                        vals[d0])
                return 0
            lax.fori_loop(0, (CHUNK // LANES) * (EMB // LANES), blkbody, 0)

        ngrp = gpw // NBUF

        for b in range(NBUF):
            fire_gather(b, b)
        for b in range(NBUF):
            wait_gather(b, b)
            transpose_scale(b)
            fire_out(b, b)
            fire_gather(b + NBUF, b)

        def group(gg, _):
            for b in range(NBUF):
                g = gg * NBUF + b
                wait_gather(g, b)
                wait_out(g - NBUF, b)
                transpose_scale(b)
                fire_out(g, b)
                fire_gather(g + NBUF, b)
            return 0
        lax.fori_loop(1, ngrp - 1, group, 0)

        for b in range(NBUF):
            g = (ngrp - 1) * NBUF + b
            wait_gather(g, b)
            wait_out(g - NBUF, b)
            transpose_scale(b)
            fire_out(g, b)
        for b in range(NBUF):
            g = (ngrp - 1) * NBUF + b
            wait_out(g, b)

    return emb


def kernel(mask, weight):
    bsz, hist = mask.shape
    btiles = bsz // CHUNK
    mask3 = jnp.transpose(mask).astype(jnp.int32).reshape(NW, (hist * btiles) // NW, CHUNK)
    vocab = weight.shape[0]
    wt2 = _build_prep(vocab)(jnp.transpose(weight))
    if vocab % CHUNK:
        vtail = (vocab // CHUNK) * CHUNK
        tail = (weight[vtail:] * jnp.float32(SCALE)).reshape(-1)
        wt2 = jax.lax.dynamic_update_slice(wt2, tail, (vtail * EMB,))
    out5 = _build(hist, btiles)(mask3, wt2.reshape(vocab, EMB))
    return out5.transpose(2, 4, 0, 1, 3).reshape(bsz, hist, EMB)
